# baseline (speedup 1.0000x reference)
import functools

import jax
import jax.numpy as jnp
from jax import lax
import jax.experimental.pallas as pl
from jax.experimental.pallas import tpu as pltpu
from jax.experimental.pallas import tpu_sc as plsc

_NW = 32
_CH = 80
_NB = 4


@functools.partial(jax.jit, static_argnames=("n_rows", "d", "ch", "nb"))
def _sc_gather(table, idx2, *, n_rows, d, ch=128, nb=6):
    per_w = n_rows // _NW
    n_s = per_w // ch
    tail = per_w - n_s * ch
    assert tail % 8 == 0 and ch % 8 == 0
    dt = table.dtype
    mesh = plsc.VectorSubcoreMesh(core_axis_name="c", subcore_axis_name="s")

    scratch = ([pltpu.VMEM((per_w,), jnp.int32)]
               + [pltpu.VMEM((ch, d), dt) for _ in range(nb)]
               + [pltpu.SemaphoreType.DMA for _ in range(2 * nb)])

    @functools.partial(
        pl.kernel, mesh=mesh,
        out_type=jax.ShapeDtypeStruct((n_rows, d), dt),
        scratch_types=scratch,
    )
    def k(table_hbm, idx_hbm, out_hbm, idx_v, *rest):
        bufs = rest[:nb]
        gsem = rest[nb:2 * nb]
        osem = rest[2 * nb:3 * nb]
        wid = lax.axis_index("s") * 2 + lax.axis_index("c")
        base = wid * per_w
        pltpu.sync_copy(idx_hbm.at[wid], idx_v)

        if tail:
            tdesc = pltpu.make_async_copy(
                table_hbm.at[idx_v.at[pl.ds(n_s * ch, tail)]],
                bufs[0].at[pl.ds(0, tail)], gsem[0])
            tdesc.start()
            tdesc.wait()
            pltpu.sync_copy(bufs[0].at[pl.ds(0, tail)],
                            out_hbm.at[pl.ds(base + n_s * ch, tail)])

        def gather_desc(s, b, sem):
            return pltpu.make_async_copy(
                table_hbm.at[idx_v.at[pl.ds(s * ch, ch)]], bufs[b], sem)

        def out_desc(s, b, sem):
            return pltpu.make_async_copy(
                bufs[b], out_hbm.at[pl.ds(base + s * ch, ch)], sem)

        gather_desc(0, 0, gsem[0]).start()

        n_outer = (n_s + nb - 1) // nb

        def outer(t, carry):
            for b in range(nb):
                s = t * nb + b

                @pl.when(s < n_s)
                def _():
                    gather_desc(s, b, gsem[b]).wait()
                    out_desc(s, b, osem[b]).start()
                    sn = s + 1
                    bn = (b + 1) % nb

                    @pl.when(sn < n_s)
                    def _():
                        @pl.when(sn >= nb)
                        def _():
                            out_desc(sn - nb, bn, osem[bn]).wait()
                        gather_desc(sn, bn, gsem[bn]).start()
            return carry

        lax.fori_loop(0, n_outer, outer, 0)
        for s in range(max(0, n_s - nb), n_s):
            out_desc(s, s % nb, osem[s % nb]).wait()

    return k(table, idx2)


def _subcore_ranges(total):
    rpt = -(-total // 16 // 8) * 8
    last = total - 15 * rpt
    assert last > 0 and last % 8 == 0
    return rpt, last


@functools.partial(jax.jit, static_argnames=("n_nodes", "d", "ch"))
def _sc_scatter_add(rows, idx6, zeros, *, n_nodes, d, ch=128):
    n_rows = rows.shape[0]
    half = n_nodes // 2
    acc_rows = half + 8
    per_t = n_rows // 16
    n_s = per_t // ch
    tail = per_t - n_s * ch
    n_chunks = n_s + (1 if tail else 0)
    rpt_i, last_i = _subcore_ranges(acc_rows)
    rpt_o, last_o = _subcore_ranges(half)
    mesh = plsc.VectorSubcoreMesh(core_axis_name="c", subcore_axis_name="s")

    scratch = ([pltpu.VMEM((n_chunks, ch), jnp.int32)]
               + [pltpu.VMEM((ch, d), jnp.float32) for _ in range(_NB)]
               + [pltpu.SemaphoreType.DMA for _ in range(_NB)]
               + [pltpu.VMEM_SHARED((acc_rows, d), jnp.float32)])

    @functools.partial(
        pl.kernel, mesh=mesh,
        out_type=jax.ShapeDtypeStruct((n_nodes, d), jnp.float32),
        scratch_types=scratch,
    )
    def k(rows_hbm, idx_hbm, zeros_hbm, out_hbm, idx_v, *rest):
        bufs = rest[:_NB]
        isem = rest[_NB:2 * _NB]
        acc = rest[2 * _NB]
        cid = lax.axis_index("c")
        sid = lax.axis_index("s")
        base = sid * per_t
        pltpu.sync_copy(idx_hbm.at[cid, sid], idx_v)

        @pl.when(sid < 15)
        def _():
            pltpu.sync_copy(zeros_hbm.at[pl.ds(sid * rpt_i, rpt_i)],
                            acc.at[pl.ds(sid * rpt_i, rpt_i)])

        @pl.when(sid == 15)
        def _():
            pltpu.sync_copy(zeros_hbm.at[pl.ds(15 * rpt_i, last_i)],
                            acc.at[pl.ds(15 * rpt_i, last_i)])

        plsc.subcore_barrier()

        if tail:
            tdesc = pltpu.make_async_copy(
                rows_hbm.at[pl.ds(base + n_s * ch, tail)],
                bufs[0].at[pl.ds(0, tail)], isem[0])
            tdesc.start()
            tdesc.wait()
            pltpu.sync_copy(bufs[0], acc.at[idx_v.at[n_s]], add=True)

        def in_desc(s, b):
            return pltpu.make_async_copy(
                rows_hbm.at[pl.ds(base + s * ch, ch)], bufs[b], isem[b])

        for b in range(_NB):
            if b < n_s:
                in_desc(b, b).start()

        n_outer = (n_s + _NB - 1) // _NB

        def outer(t, carry):
            for b in range(_NB):
                s = t * _NB + b

                @pl.when(s < n_s)
                def _():
                    in_desc(s, b).wait()
                    pltpu.sync_copy(bufs[b], acc.at[idx_v.at[s]], add=True)
                    sn = s + _NB

                    @pl.when(sn < n_s)
                    def _():
                        in_desc(sn, b).start()
            return carry

        lax.fori_loop(0, n_outer, outer, 0)
        plsc.subcore_barrier()

        @pl.when(sid < 15)
        def _():
            pltpu.sync_copy(acc.at[pl.ds(sid * rpt_o, rpt_o)],
                            out_hbm.at[pl.ds(cid * half + sid * rpt_o, rpt_o)])

        @pl.when(sid == 15)
        def _():
            pltpu.sync_copy(acc.at[pl.ds(15 * rpt_o, last_o)],
                            out_hbm.at[pl.ds(cid * half + 15 * rpt_o, last_o)])

    return k(rows, idx6, zeros)


@functools.partial(jax.jit, static_argnames=("n_rows", "n_nodes", "d", "ch"))
def _sc_count(idx6, ones_rows, zeros, *, n_rows, n_nodes, d, ch=128):
    half = n_nodes // 2
    acc_rows = half + 8
    per_t = n_rows // 16
    n_s = -(-per_t // ch)
    rpt_i, last_i = _subcore_ranges(acc_rows)
    rpt_o, last_o = _subcore_ranges(half)
    mesh = plsc.VectorSubcoreMesh(core_axis_name="c", subcore_axis_name="s")

    scratch = [pltpu.VMEM((n_s, ch), jnp.int32),
               pltpu.VMEM((ch, d), jnp.float32),
               pltpu.VMEM_SHARED((acc_rows, d), jnp.float32)]

    @functools.partial(
        pl.kernel, mesh=mesh,
        out_type=jax.ShapeDtypeStruct((n_nodes, d), jnp.float32),
        scratch_types=scratch,
    )
    def k(idx_hbm, ones_hbm, zeros_hbm, out_hbm, idx_v, ones_v, acc):
        cid = lax.axis_index("c")
        sid = lax.axis_index("s")
        pltpu.sync_copy(idx_hbm.at[cid, sid], idx_v)
        pltpu.sync_copy(ones_hbm, ones_v)

        @pl.when(sid < 15)
        def _():
            pltpu.sync_copy(zeros_hbm.at[pl.ds(sid * rpt_i, rpt_i)],
                            acc.at[pl.ds(sid * rpt_i, rpt_i)])

        @pl.when(sid == 15)
        def _():
            pltpu.sync_copy(zeros_hbm.at[pl.ds(15 * rpt_i, last_i)],
                            acc.at[pl.ds(15 * rpt_i, last_i)])

        plsc.subcore_barrier()

        def body(s, carry):
            pltpu.sync_copy(ones_v, acc.at[idx_v.at[s]], add=True)
            return carry

        lax.fori_loop(0, n_s, body, 0)
        plsc.subcore_barrier()

        @pl.when(sid < 15)
        def _():
            pltpu.sync_copy(acc.at[pl.ds(sid * rpt_o, rpt_o)],
                            out_hbm.at[pl.ds(cid * half + sid * rpt_o, rpt_o)])

        @pl.when(sid == 15)
        def _():
            pltpu.sync_copy(acc.at[pl.ds(15 * rpt_o, last_o)],
                            out_hbm.at[pl.ds(cid * half + 15 * rpt_o, last_o)])

    return k(idx6, ones_rows, zeros)


def _tc_dual_proj(h, wa, wb):
    n, din = h.shape
    da = wa.shape[1]
    db = wb.shape[1]
    blk = 2000

    def body(h_ref, wa_ref, wb_ref, oa_ref, ob_ref):
        hb = h_ref[...]
        oa_ref[...] = jnp.dot(hb, wa_ref[...],
                              preferred_element_type=jnp.float32)
        ob_ref[...] = jnp.dot(hb, wb_ref[...],
                              preferred_element_type=jnp.float32)

    return pl.pallas_call(
        body,
        grid=(n // blk,),
        in_specs=[pl.BlockSpec((blk, din), lambda i: (i, 0)),
                  pl.BlockSpec((din, da), lambda i: (0, 0)),
                  pl.BlockSpec((din, db), lambda i: (0, 0))],
        out_specs=[pl.BlockSpec((blk, da), lambda i: (i, 0)),
                   pl.BlockSpec((blk, db), lambda i: (i, 0))],
        out_shape=[jax.ShapeDtypeStruct((n, da), jnp.float32),
                   jax.ShapeDtypeStruct((n, db), jnp.float32)],
    )(h, wa, wb)


def _tc_msg_mlp(gd, gs, e, w1e, b1, w2, b2, w3, b3):
    n, d = gd.shape
    de = e.shape[1]
    blk = 2000

    def body(gd_ref, gs_ref, e_ref, w1e_ref, b1_ref, w2_ref, b2_ref,
             w3_ref, b3_ref, o_ref):
        t = gd_ref[...] + gs_ref[...] + jnp.dot(
            e_ref[...], w1e_ref[...], preferred_element_type=jnp.float32)
        t = jnp.maximum(t + b1_ref[...], 0.0).astype(jnp.bfloat16)
        t = jnp.maximum(
            jnp.dot(t, w2_ref[...].astype(jnp.bfloat16),
                    preferred_element_type=jnp.float32)
            + b2_ref[...], 0.0).astype(jnp.bfloat16)
        o_ref[...] = jnp.dot(
            t, w3_ref[...].astype(jnp.bfloat16),
            preferred_element_type=jnp.float32) + b3_ref[...]

    full = pl.BlockSpec((1, d), lambda i: (0, 0))
    return pl.pallas_call(
        body,
        grid=(n // blk,),
        in_specs=[pl.BlockSpec((blk, d), lambda i: (i, 0)),
                  pl.BlockSpec((blk, d), lambda i: (i, 0)),
                  pl.BlockSpec((blk, de), lambda i: (i, 0)),
                  pl.BlockSpec((de, d), lambda i: (0, 0)),
                  full,
                  pl.BlockSpec((d, d), lambda i: (0, 0)),
                  full,
                  pl.BlockSpec((d, d), lambda i: (0, 0)),
                  full],
        out_specs=pl.BlockSpec((blk, d), lambda i: (i, 0)),
        out_shape=jax.ShapeDtypeStruct((n, d), jnp.float32),
    )(gd, gs, e, w1e, b1.reshape(1, d), w2, b2.reshape(1, d),
      w3, b3.reshape(1, d))


def _layer_norm(v, g, b):
    mu = jnp.mean(v, axis=-1, keepdims=True)
    var = jnp.mean((v - mu) ** 2, axis=-1, keepdims=True)
    return (v - mu) * lax.rsqrt(var + 1e-5) * g + b


def _tc_node_update(h, agg_in, counts, ln1g, ln1b, wf1, bf1, wf2,
                    bf2, ln2g, ln2b):
    n, d = h.shape
    dh = wf1.shape[1]
    blk = 2000

    def body(h_ref, pa_ref, c_ref, ln1g_ref, ln1b_ref, wf1_ref,
             bf1_ref, wf2_ref, bf2_ref, ln2g_ref, ln2b_ref, o_ref):
        cnt = c_ref[...][:, 0:1]
        agg = pa_ref[...] / jnp.maximum(cnt, 1.0)
        u = _layer_norm(h_ref[...] + agg, ln1g_ref[...], ln1b_ref[...])
        ff = jnp.maximum(
            jnp.dot(u, wf1_ref[...], preferred_element_type=jnp.float32)
            + bf1_ref[...], 0.0)
        ff = jnp.dot(ff, wf2_ref[...], preferred_element_type=jnp.float32) \
            + bf2_ref[...]
        o_ref[...] = _layer_norm(u + ff, ln2g_ref[...], ln2b_ref[...])

    vec = pl.BlockSpec((1, d), lambda i: (0, 0))
    return pl.pallas_call(
        body,
        grid=(n // blk,),
        in_specs=[pl.BlockSpec((blk, d), lambda i: (i, 0)),
                  pl.BlockSpec((blk, d), lambda i: (i, 0)),
                  pl.BlockSpec((blk, d), lambda i: (i, 0)),
                  vec, vec,
                  pl.BlockSpec((d, dh), lambda i: (0, 0)),
                  pl.BlockSpec((1, dh), lambda i: (0, 0)),
                  pl.BlockSpec((dh, d), lambda i: (0, 0)),
                  vec, vec, vec],
        out_specs=pl.BlockSpec((blk, d), lambda i: (i, 0)),
        out_shape=jax.ShapeDtypeStruct((n, d), jnp.float32),
    )(h, agg_in, counts, ln1g.reshape(1, d), ln1b.reshape(1, d), wf1,
      bf1.reshape(1, dh), wf2, bf2.reshape(1, d), ln2g.reshape(1, d),
      ln2b.reshape(1, d))


def _tc_boundary_msg(hd, hs, e, eparams, lne, mparams):
    n, d = hd.shape
    de = e.shape[1]
    blk = 2000
    (u1, c1), (u2, c2), (u3, c3) = eparams
    lng, lnb = lne
    (w1, b1), (w2, b2), (w3, b3) = mparams
    u1s, u1d, u1e = u1[:d], u1[d:2 * d], u1[2 * d:]
    w1d, w1e, w1s = w1[:d], w1[d:d + de], w1[d + de:]

    def body(hd_ref, hs_ref, e_ref, u1s_ref, u1d_ref, u1e_ref, c1_ref,
             u2_ref, c2_ref, u3_ref, c3_ref, lng_ref, lnb_ref,
             w1d_ref, w1e_ref, w1s_ref, b1_ref, w2_ref, b2_ref,
             w3_ref, b3_ref, o_ref):
        hdv = hd_ref[...]
        hsv = hs_ref[...]
        ev = e_ref[...]

        def mm(a, w_ref):
            return jnp.dot(a, w_ref[...], preferred_element_type=jnp.float32)

        t = mm(hsv, u1s_ref) + mm(hdv, u1d_ref) + mm(ev, u1e_ref)
        t = jnp.maximum(t + c1_ref[...], 0.0)
        t = jnp.maximum(mm(t, u2_ref) + c2_ref[...], 0.0)
        t = mm(t, u3_ref) + c3_ref[...]
        e2 = _layer_norm(ev + t, lng_ref[...], lnb_ref[...])

        def mmb(a, w_ref):
            return jnp.dot(a.astype(jnp.bfloat16),
                           w_ref[...].astype(jnp.bfloat16),
                           preferred_element_type=jnp.float32)

        m = mmb(hdv, w1d_ref) + mmb(hsv, w1s_ref) + mm(e2, w1e_ref)
        m = jnp.maximum(m + b1_ref[...], 0.0)
        m = jnp.maximum(mmb(m, w2_ref) + b2_ref[...], 0.0)
        o_ref[...] = mmb(m, w3_ref) + b3_ref[...]

    hb = pl.BlockSpec((blk, d), lambda i: (i, 0))
    eb = pl.BlockSpec((blk, de), lambda i: (i, 0))
    p128 = pl.BlockSpec((d, 128), lambda i: (0, 0))
    p16 = pl.BlockSpec((d, de), lambda i: (0, 0))
    sq = pl.BlockSpec((de, de), lambda i: (0, 0))
    v16 = pl.BlockSpec((1, de), lambda i: (0, 0))
    v128 = pl.BlockSpec((1, d), lambda i: (0, 0))
    return pl.pallas_call(
        body,
        grid=(n // blk,),
        in_specs=[hb, hb, eb,
                  p16, p16, sq, v16,
                  sq, v16, sq, v16,
                  v16, v16,
                  p128, pl.BlockSpec((de, d), lambda i: (0, 0)), p128, v128,
                  pl.BlockSpec((d, d), lambda i: (0, 0)), v128,
                  pl.BlockSpec((d, d), lambda i: (0, 0)), v128],
        out_specs=hb,
        out_shape=jax.ShapeDtypeStruct((n, d), jnp.float32),
    )(hd, hs, e, u1s, u1d, u1e, c1.reshape(1, de), u2, c2.reshape(1, de),
      u3, c3.reshape(1, de), lng.reshape(1, de), lnb.reshape(1, de),
      w1d, w1e, w1s, b1.reshape(1, d), w2, b2.reshape(1, d),
      w3, b3.reshape(1, d))


def kernel(x, edge_index, edge_attr, params):
    n_nodes = x.shape[1]
    d = x.shape[2]
    de = edge_attr.shape[1]
    n_edges = edge_index.shape[1]
    per_w = n_edges // _NW
    n_s = per_w // _CH

    h = x.reshape(n_nodes, d)
    e = edge_attr
    src = edge_index[0].astype(jnp.int32)
    dst = edge_index[1].astype(jnp.int32)

    per_w = n_edges // _NW
    src2 = src.reshape(_NW, per_w)
    dst2 = dst.reshape(_NW, per_w)

    half = n_nodes // 2
    dst_lo = jnp.where(dst < half, dst, half)
    dst_hi = jnp.where(dst >= half, dst - half, half)
    sch = 128
    per_t = n_edges // 16
    n_chunks = -(-per_t // sch)
    pad = n_chunks * sch - per_t
    idx6 = jnp.stack([dst_lo, dst_hi]).reshape(2, 16, per_t)
    idx6 = jnp.pad(idx6, ((0, 0), (0, 0), (0, pad)), constant_values=half)
    idx6 = idx6.reshape(2, 16, n_chunks, sch)
    zeros_acc = jnp.zeros((half + 8, d), jnp.float32)
    counts = _sc_count(idx6, jnp.ones((sch, d), jnp.float32), zeros_acc,
                       n_rows=n_edges, n_nodes=n_nodes, d=d, ch=sch)

    layers = params["layers"]
    n_layers = len(layers)
    mrows = None
    for li, p in enumerate(layers):
        if mrows is None:
            (w1, b1), (w2, b2), (w3, b3) = p["msg"]
            w1d, w1e, w1s = w1[:d], w1[d:d + de], w1[d + de:]
            hd, hs = _tc_dual_proj(h, w1d, w1s)
            gd = _sc_gather(hd, dst2, n_rows=n_edges, d=d)
            gs = _sc_gather(hs, src2, n_rows=n_edges, d=d)
            mrows = _tc_msg_mlp(gd, gs, e, w1e, b1, w2, b2, w3, b3)
        agg = _sc_scatter_add(mrows, idx6, zeros_acc, n_nodes=n_nodes, d=d)
        (wf1, bf1), (wf2, bf2) = p["ff"]
        h = _tc_node_update(h, agg, counts, p["ln1"][0], p["ln1"][1],
                            wf1, bf1, wf2, bf2, p["ln2"][0], p["ln2"][1])
        if li < n_layers - 1:
            rd = _sc_gather(h, dst2, n_rows=n_edges, d=d)
            rs = _sc_gather(h, src2, n_rows=n_edges, d=d)
            mrows = _tc_boundary_msg(rd, rs, e, p["edge"], p["ln_e"],
                                     layers[li + 1]["msg"])
    return h.reshape(x.shape)


# speedup vs baseline: 1.0450x; 1.0450x over previous
import functools

import jax
import jax.numpy as jnp
from jax import lax
import jax.experimental.pallas as pl
from jax.experimental.pallas import tpu as pltpu
from jax.experimental.pallas import tpu_sc as plsc

_NW = 32
_CH = 80
_NB = 4


@functools.partial(jax.jit, static_argnames=("n_rows", "d", "ch", "nb"))
def _sc_gather(table, idx2, *, n_rows, d, ch=128, nb=6):
    per_w = n_rows // _NW
    n_s = per_w // ch
    tail = per_w - n_s * ch
    assert tail % 8 == 0 and ch % 8 == 0
    dt = table.dtype
    mesh = plsc.VectorSubcoreMesh(core_axis_name="c", subcore_axis_name="s")

    scratch = ([pltpu.VMEM((per_w,), jnp.int32)]
               + [pltpu.VMEM((ch, d), dt) for _ in range(nb)]
               + [pltpu.SemaphoreType.DMA for _ in range(2 * nb)])

    @functools.partial(
        pl.kernel, mesh=mesh,
        out_type=jax.ShapeDtypeStruct((n_rows, d), dt),
        scratch_types=scratch,
    )
    def k(table_hbm, idx_hbm, out_hbm, idx_v, *rest):
        bufs = rest[:nb]
        gsem = rest[nb:2 * nb]
        osem = rest[2 * nb:3 * nb]
        wid = lax.axis_index("s") * 2 + lax.axis_index("c")
        base = wid * per_w
        pltpu.sync_copy(idx_hbm.at[wid], idx_v)

        if tail:
            tdesc = pltpu.make_async_copy(
                table_hbm.at[idx_v.at[pl.ds(n_s * ch, tail)]],
                bufs[0].at[pl.ds(0, tail)], gsem[0])
            tdesc.start()
            tdesc.wait()
            pltpu.sync_copy(bufs[0].at[pl.ds(0, tail)],
                            out_hbm.at[pl.ds(base + n_s * ch, tail)])

        def gather_desc(s, b, sem):
            return pltpu.make_async_copy(
                table_hbm.at[idx_v.at[pl.ds(s * ch, ch)]], bufs[b], sem)

        def out_desc(s, b, sem):
            return pltpu.make_async_copy(
                bufs[b], out_hbm.at[pl.ds(base + s * ch, ch)], sem)

        gather_desc(0, 0, gsem[0]).start()

        n_outer = (n_s + nb - 1) // nb

        def outer(t, carry):
            for b in range(nb):
                s = t * nb + b

                @pl.when(s < n_s)
                def _():
                    gather_desc(s, b, gsem[b]).wait()
                    out_desc(s, b, osem[b]).start()
                    sn = s + 1
                    bn = (b + 1) % nb

                    @pl.when(sn < n_s)
                    def _():
                        @pl.when(sn >= nb)
                        def _():
                            out_desc(sn - nb, bn, osem[bn]).wait()
                        gather_desc(sn, bn, gsem[bn]).start()
            return carry

        lax.fori_loop(0, n_outer, outer, 0)
        for s in range(max(0, n_s - nb), n_s):
            out_desc(s, s % nb, osem[s % nb]).wait()

    return k(table, idx2)


def _subcore_ranges(total):
    rpt = -(-total // 16 // 8) * 8
    last = total - 15 * rpt
    assert last > 0 and last % 8 == 0
    return rpt, last


@functools.partial(jax.jit, static_argnames=("n_nodes", "d", "ch"))
def _sc_scatter_add(rows, idx6, zeros, *, n_nodes, d, ch=128):
    n_rows = rows.shape[0]
    half = n_nodes // 2
    acc_rows = half + 8
    per_t = n_rows // 16
    n_s = per_t // ch
    tail = per_t - n_s * ch
    n_chunks = n_s + (1 if tail else 0)
    rpt_i, last_i = _subcore_ranges(acc_rows)
    rpt_o, last_o = _subcore_ranges(half)
    mesh = plsc.VectorSubcoreMesh(core_axis_name="c", subcore_axis_name="s")

    scratch = ([pltpu.VMEM((n_chunks, ch), jnp.int32)]
               + [pltpu.VMEM((ch, d), jnp.float32) for _ in range(_NB)]
               + [pltpu.SemaphoreType.DMA for _ in range(_NB)]
               + [pltpu.VMEM_SHARED((acc_rows, d), jnp.float32)])

    @functools.partial(
        pl.kernel, mesh=mesh,
        out_type=jax.ShapeDtypeStruct((n_nodes, d), jnp.float32),
        scratch_types=scratch,
    )
    def k(rows_hbm, idx_hbm, zeros_hbm, out_hbm, idx_v, *rest):
        bufs = rest[:_NB]
        isem = rest[_NB:2 * _NB]
        acc = rest[2 * _NB]
        cid = lax.axis_index("c")
        sid = lax.axis_index("s")
        base = sid * per_t
        pltpu.sync_copy(idx_hbm.at[cid, sid], idx_v)

        @pl.when(sid < 15)
        def _():
            pltpu.sync_copy(zeros_hbm.at[pl.ds(sid * rpt_i, rpt_i)],
                            acc.at[pl.ds(sid * rpt_i, rpt_i)])

        @pl.when(sid == 15)
        def _():
            pltpu.sync_copy(zeros_hbm.at[pl.ds(15 * rpt_i, last_i)],
                            acc.at[pl.ds(15 * rpt_i, last_i)])

        plsc.subcore_barrier()

        if tail:
            tdesc = pltpu.make_async_copy(
                rows_hbm.at[pl.ds(base + n_s * ch, tail)],
                bufs[0].at[pl.ds(0, tail)], isem[0])
            tdesc.start()
            tdesc.wait()
            pltpu.sync_copy(bufs[0], acc.at[idx_v.at[n_s]], add=True)

        def in_desc(s, b):
            return pltpu.make_async_copy(
                rows_hbm.at[pl.ds(base + s * ch, ch)], bufs[b], isem[b])

        for b in range(_NB):
            if b < n_s:
                in_desc(b, b).start()

        n_outer = (n_s + _NB - 1) // _NB

        def outer(t, carry):
            for b in range(_NB):
                s = t * _NB + b

                @pl.when(s < n_s)
                def _():
                    in_desc(s, b).wait()
                    pltpu.sync_copy(bufs[b], acc.at[idx_v.at[s]], add=True)
                    sn = s + _NB

                    @pl.when(sn < n_s)
                    def _():
                        in_desc(sn, b).start()
            return carry

        lax.fori_loop(0, n_outer, outer, 0)
        plsc.subcore_barrier()

        @pl.when(sid < 15)
        def _():
            pltpu.sync_copy(acc.at[pl.ds(sid * rpt_o, rpt_o)],
                            out_hbm.at[pl.ds(cid * half + sid * rpt_o, rpt_o)])

        @pl.when(sid == 15)
        def _():
            pltpu.sync_copy(acc.at[pl.ds(15 * rpt_o, last_o)],
                            out_hbm.at[pl.ds(cid * half + 15 * rpt_o, last_o)])

    return k(rows, idx6, zeros)


@functools.partial(jax.jit, static_argnames=("n_rows", "n_nodes", "d", "ch"))
def _sc_count(idx6, ones_rows, zeros, *, n_rows, n_nodes, d, ch=128):
    half = n_nodes // 2
    acc_rows = half + 8
    per_t = n_rows // 16
    n_s = -(-per_t // ch)
    rpt_i, last_i = _subcore_ranges(acc_rows)
    rpt_o, last_o = _subcore_ranges(half)
    mesh = plsc.VectorSubcoreMesh(core_axis_name="c", subcore_axis_name="s")

    scratch = [pltpu.VMEM((n_s, ch), jnp.int32),
               pltpu.VMEM((ch, d), jnp.float32),
               pltpu.VMEM_SHARED((acc_rows, d), jnp.float32)]

    @functools.partial(
        pl.kernel, mesh=mesh,
        out_type=jax.ShapeDtypeStruct((n_nodes, d), jnp.float32),
        scratch_types=scratch,
    )
    def k(idx_hbm, ones_hbm, zeros_hbm, out_hbm, idx_v, ones_v, acc):
        cid = lax.axis_index("c")
        sid = lax.axis_index("s")
        pltpu.sync_copy(idx_hbm.at[cid, sid], idx_v)
        pltpu.sync_copy(ones_hbm, ones_v)

        @pl.when(sid < 15)
        def _():
            pltpu.sync_copy(zeros_hbm.at[pl.ds(sid * rpt_i, rpt_i)],
                            acc.at[pl.ds(sid * rpt_i, rpt_i)])

        @pl.when(sid == 15)
        def _():
            pltpu.sync_copy(zeros_hbm.at[pl.ds(15 * rpt_i, last_i)],
                            acc.at[pl.ds(15 * rpt_i, last_i)])

        plsc.subcore_barrier()

        def body(s, carry):
            pltpu.sync_copy(ones_v, acc.at[idx_v.at[s]], add=True)
            return carry

        lax.fori_loop(0, n_s, body, 0)
        plsc.subcore_barrier()

        @pl.when(sid < 15)
        def _():
            pltpu.sync_copy(acc.at[pl.ds(sid * rpt_o, rpt_o)],
                            out_hbm.at[pl.ds(cid * half + sid * rpt_o, rpt_o)])

        @pl.when(sid == 15)
        def _():
            pltpu.sync_copy(acc.at[pl.ds(15 * rpt_o, last_o)],
                            out_hbm.at[pl.ds(cid * half + 15 * rpt_o, last_o)])

    return k(idx6, ones_rows, zeros)


def _tc_dual_proj(h, wa, wb):
    n, din = h.shape
    da = wa.shape[1]
    db = wb.shape[1]
    blk = 2000

    def body(h_ref, wa_ref, wb_ref, oa_ref, ob_ref):
        hb = h_ref[...]
        oa_ref[...] = jnp.dot(hb, wa_ref[...],
                              preferred_element_type=jnp.float32)
        ob_ref[...] = jnp.dot(hb, wb_ref[...],
                              preferred_element_type=jnp.float32)

    return pl.pallas_call(
        body,
        grid=(n // blk,),
        in_specs=[pl.BlockSpec((blk, din), lambda i: (i, 0)),
                  pl.BlockSpec((din, da), lambda i: (0, 0)),
                  pl.BlockSpec((din, db), lambda i: (0, 0))],
        out_specs=[pl.BlockSpec((blk, da), lambda i: (i, 0)),
                   pl.BlockSpec((blk, db), lambda i: (i, 0))],
        out_shape=[jax.ShapeDtypeStruct((n, da), jnp.float32),
                   jax.ShapeDtypeStruct((n, db), jnp.float32)],
    )(h, wa, wb)


def _tc_msg_mlp(gd, gs, e, w1e, b1, w2, b2, w3, b3):
    n, d = gd.shape
    de = e.shape[1]
    blk = 2000

    def body(gd_ref, gs_ref, e_ref, w1e_ref, b1_ref, w2_ref, b2_ref,
             w3_ref, b3_ref, o_ref):
        t = gd_ref[...] + gs_ref[...] + jnp.dot(
            e_ref[...], w1e_ref[...], preferred_element_type=jnp.float32)
        t = jnp.maximum(t + b1_ref[...], 0.0)
        t = jnp.maximum(
            jnp.dot(t, w2_ref[...], preferred_element_type=jnp.float32)
            + b2_ref[...], 0.0)
        o_ref[...] = jnp.dot(
            t, w3_ref[...], preferred_element_type=jnp.float32) + b3_ref[...]

    full = pl.BlockSpec((1, d), lambda i: (0, 0))
    return pl.pallas_call(
        body,
        grid=(n // blk,),
        in_specs=[pl.BlockSpec((blk, d), lambda i: (i, 0)),
                  pl.BlockSpec((blk, d), lambda i: (i, 0)),
                  pl.BlockSpec((blk, de), lambda i: (i, 0)),
                  pl.BlockSpec((de, d), lambda i: (0, 0)),
                  full,
                  pl.BlockSpec((d, d), lambda i: (0, 0)),
                  full,
                  pl.BlockSpec((d, d), lambda i: (0, 0)),
                  full],
        out_specs=pl.BlockSpec((blk, d), lambda i: (i, 0)),
        out_shape=jax.ShapeDtypeStruct((n, d), jnp.float32),
    )(gd, gs, e, w1e, b1.reshape(1, d), w2, b2.reshape(1, d),
      w3, b3.reshape(1, d))


def _layer_norm(v, g, b):
    mu = jnp.mean(v, axis=-1, keepdims=True)
    var = jnp.mean((v - mu) ** 2, axis=-1, keepdims=True)
    return (v - mu) * lax.rsqrt(var + 1e-5) * g + b


def _tc_node_update(h, agg_in, counts, ln1g, ln1b, wf1, bf1, wf2,
                    bf2, ln2g, ln2b):
    n, d = h.shape
    dh = wf1.shape[1]
    blk = 2000

    def body(h_ref, pa_ref, c_ref, ln1g_ref, ln1b_ref, wf1_ref,
             bf1_ref, wf2_ref, bf2_ref, ln2g_ref, ln2b_ref, o_ref):
        cnt = c_ref[...][:, 0:1]
        agg = pa_ref[...] / jnp.maximum(cnt, 1.0)
        u = _layer_norm(h_ref[...] + agg, ln1g_ref[...], ln1b_ref[...])
        ff = jnp.maximum(
            jnp.dot(u, wf1_ref[...], preferred_element_type=jnp.float32)
            + bf1_ref[...], 0.0)
        ff = jnp.dot(ff, wf2_ref[...], preferred_element_type=jnp.float32) \
            + bf2_ref[...]
        o_ref[...] = _layer_norm(u + ff, ln2g_ref[...], ln2b_ref[...])

    vec = pl.BlockSpec((1, d), lambda i: (0, 0))
    return pl.pallas_call(
        body,
        grid=(n // blk,),
        in_specs=[pl.BlockSpec((blk, d), lambda i: (i, 0)),
                  pl.BlockSpec((blk, d), lambda i: (i, 0)),
                  pl.BlockSpec((blk, d), lambda i: (i, 0)),
                  vec, vec,
                  pl.BlockSpec((d, dh), lambda i: (0, 0)),
                  pl.BlockSpec((1, dh), lambda i: (0, 0)),
                  pl.BlockSpec((dh, d), lambda i: (0, 0)),
                  vec, vec, vec],
        out_specs=pl.BlockSpec((blk, d), lambda i: (i, 0)),
        out_shape=jax.ShapeDtypeStruct((n, d), jnp.float32),
    )(h, agg_in, counts, ln1g.reshape(1, d), ln1b.reshape(1, d), wf1,
      bf1.reshape(1, dh), wf2, bf2.reshape(1, d), ln2g.reshape(1, d),
      ln2b.reshape(1, d))


def _tc_boundary_msg(hd, hs, e, eparams, lne, mparams):
    n, d = hd.shape
    de = e.shape[1]
    blk = 2000
    (u1, c1), (u2, c2), (u3, c3) = eparams
    lng, lnb = lne
    (w1, b1), (w2, b2), (w3, b3) = mparams
    u1s, u1d, u1e = u1[:d], u1[d:2 * d], u1[2 * d:]
    w1d, w1e, w1s = w1[:d], w1[d:d + de], w1[d + de:]

    def body(hd_ref, hs_ref, e_ref, u1s_ref, u1d_ref, u1e_ref, c1_ref,
             u2_ref, c2_ref, u3_ref, c3_ref, lng_ref, lnb_ref,
             w1d_ref, w1e_ref, w1s_ref, b1_ref, w2_ref, b2_ref,
             w3_ref, b3_ref, o_ref):
        hdv = hd_ref[...]
        hsv = hs_ref[...]
        ev = e_ref[...]

        def mm(a, w_ref):
            return jnp.dot(a, w_ref[...], preferred_element_type=jnp.float32)

        t = mm(hsv, u1s_ref) + mm(hdv, u1d_ref) + mm(ev, u1e_ref)
        t = jnp.maximum(t + c1_ref[...], 0.0)
        t = jnp.maximum(mm(t, u2_ref) + c2_ref[...], 0.0)
        t = mm(t, u3_ref) + c3_ref[...]
        e2 = _layer_norm(ev + t, lng_ref[...], lnb_ref[...])

        m = mm(hdv, w1d_ref) + mm(hsv, w1s_ref) + mm(e2, w1e_ref)
        m = jnp.maximum(m + b1_ref[...], 0.0)
        m = jnp.maximum(mm(m, w2_ref) + b2_ref[...], 0.0)
        o_ref[...] = mm(m, w3_ref) + b3_ref[...]

    hb = pl.BlockSpec((blk, d), lambda i: (i, 0))
    eb = pl.BlockSpec((blk, de), lambda i: (i, 0))
    p128 = pl.BlockSpec((d, 128), lambda i: (0, 0))
    p16 = pl.BlockSpec((d, de), lambda i: (0, 0))
    sq = pl.BlockSpec((de, de), lambda i: (0, 0))
    v16 = pl.BlockSpec((1, de), lambda i: (0, 0))
    v128 = pl.BlockSpec((1, d), lambda i: (0, 0))
    return pl.pallas_call(
        body,
        grid=(n // blk,),
        in_specs=[hb, hb, eb,
                  p16, p16, sq, v16,
                  sq, v16, sq, v16,
                  v16, v16,
                  p128, pl.BlockSpec((de, d), lambda i: (0, 0)), p128, v128,
                  pl.BlockSpec((d, d), lambda i: (0, 0)), v128,
                  pl.BlockSpec((d, d), lambda i: (0, 0)), v128],
        out_specs=hb,
        out_shape=jax.ShapeDtypeStruct((n, d), jnp.float32),
    )(hd, hs, e, u1s, u1d, u1e, c1.reshape(1, de), u2, c2.reshape(1, de),
      u3, c3.reshape(1, de), lng.reshape(1, de), lnb.reshape(1, de),
      w1d, w1e, w1s, b1.reshape(1, d), w2, b2.reshape(1, d),
      w3, b3.reshape(1, d))


def kernel(x, edge_index, edge_attr, params):
    n_nodes = x.shape[1]
    d = x.shape[2]
    de = edge_attr.shape[1]
    n_edges = edge_index.shape[1]
    per_w = n_edges // _NW
    n_s = per_w // _CH

    h = x.reshape(n_nodes, d)
    e = edge_attr
    src = edge_index[0].astype(jnp.int32)
    dst = edge_index[1].astype(jnp.int32)

    per_w = n_edges // _NW
    src2 = src.reshape(_NW, per_w)
    dst2 = dst.reshape(_NW, per_w)

    half = n_nodes // 2
    dst_lo = jnp.where(dst < half, dst, half)
    dst_hi = jnp.where(dst >= half, dst - half, half)
    sch = 128
    per_t = n_edges // 16
    n_chunks = -(-per_t // sch)
    pad = n_chunks * sch - per_t
    idx6 = jnp.stack([dst_lo, dst_hi]).reshape(2, 16, per_t)
    idx6 = jnp.pad(idx6, ((0, 0), (0, 0), (0, pad)), constant_values=half)
    idx6 = idx6.reshape(2, 16, n_chunks, sch)
    zeros_acc = jnp.zeros((half + 8, d), jnp.float32)

    layers = params["layers"]
    n_layers = len(layers)
    mrows = None
    for li, p in enumerate(layers):
        if mrows is None:
            (w1, b1), (w2, b2), (w3, b3) = p["msg"]
            w1d, w1e, w1s = w1[:d], w1[d:d + de], w1[d + de:]
            hd, hs = _tc_dual_proj(h, w1d, w1s)
            gd = _sc_gather(hd, dst2, n_rows=n_edges, d=d)
            gs = _sc_gather(hs, src2, n_rows=n_edges, d=d)
            idx6_b, _ = lax.optimization_barrier((idx6, gs))
            counts = _sc_count(idx6_b, jnp.ones((sch, d), jnp.float32),
                               zeros_acc, n_rows=n_edges, n_nodes=n_nodes,
                               d=d, ch=sch)
            mrows = _tc_msg_mlp(gd, gs, e, w1e, b1, w2, b2, w3, b3)
        agg = _sc_scatter_add(mrows, idx6, zeros_acc, n_nodes=n_nodes, d=d)
        (wf1, bf1), (wf2, bf2) = p["ff"]
        h = _tc_node_update(h, agg, counts, p["ln1"][0], p["ln1"][1],
                            wf1, bf1, wf2, bf2, p["ln2"][0], p["ln2"][1])
        if li < n_layers - 1:
            rd = _sc_gather(h, dst2, n_rows=n_edges, d=d)
            rs = _sc_gather(h, src2, n_rows=n_edges, d=d)
            mrows = _tc_boundary_msg(rd, rs, e, p["edge"], p["ln_e"],
                                     layers[li + 1]["msg"])
    return h.reshape(x.shape)


# speedup vs baseline: 1.1424x; 1.0932x over previous
import functools

import jax
import jax.numpy as jnp
from jax import lax
import jax.experimental.pallas as pl
from jax.experimental.pallas import tpu as pltpu
from jax.experimental.pallas import tpu_sc as plsc

_NW = 32
_CH = 80
_NB = 4


@functools.partial(jax.jit, static_argnames=("n_rows", "d", "ch", "nb"))
def _sc_gather(table, idx2, *, n_rows, d, ch=128, nb=6):
    per_w = n_rows // _NW
    n_s = per_w // ch
    tail = per_w - n_s * ch
    assert tail % 8 == 0 and ch % 8 == 0
    dt = table.dtype
    mesh = plsc.VectorSubcoreMesh(core_axis_name="c", subcore_axis_name="s")

    scratch = ([pltpu.VMEM((per_w,), jnp.int32)]
               + [pltpu.VMEM((ch, d), dt) for _ in range(nb)]
               + [pltpu.SemaphoreType.DMA for _ in range(2 * nb)])

    @functools.partial(
        pl.kernel, mesh=mesh,
        out_type=jax.ShapeDtypeStruct((n_rows, d), dt),
        scratch_types=scratch,
    )
    def k(table_hbm, idx_hbm, out_hbm, idx_v, *rest):
        bufs = rest[:nb]
        gsem = rest[nb:2 * nb]
        osem = rest[2 * nb:3 * nb]
        wid = lax.axis_index("s") * 2 + lax.axis_index("c")
        base = wid * per_w
        pltpu.sync_copy(idx_hbm.at[wid], idx_v)

        if tail:
            tdesc = pltpu.make_async_copy(
                table_hbm.at[idx_v.at[pl.ds(n_s * ch, tail)]],
                bufs[0].at[pl.ds(0, tail)], gsem[0])
            tdesc.start()
            tdesc.wait()
            pltpu.sync_copy(bufs[0].at[pl.ds(0, tail)],
                            out_hbm.at[pl.ds(base + n_s * ch, tail)])

        def gather_desc(s, b, sem):
            return pltpu.make_async_copy(
                table_hbm.at[idx_v.at[pl.ds(s * ch, ch)]], bufs[b], sem)

        def out_desc(s, b, sem):
            return pltpu.make_async_copy(
                bufs[b], out_hbm.at[pl.ds(base + s * ch, ch)], sem)

        gather_desc(0, 0, gsem[0]).start()

        n_outer = (n_s + nb - 1) // nb

        def outer(t, carry):
            for b in range(nb):
                s = t * nb + b

                @pl.when(s < n_s)
                def _():
                    gather_desc(s, b, gsem[b]).wait()
                    out_desc(s, b, osem[b]).start()
                    sn = s + 1
                    bn = (b + 1) % nb

                    @pl.when(sn < n_s)
                    def _():
                        @pl.when(sn >= nb)
                        def _():
                            out_desc(sn - nb, bn, osem[bn]).wait()
                        gather_desc(sn, bn, gsem[bn]).start()
            return carry

        lax.fori_loop(0, n_outer, outer, 0)
        for s in range(max(0, n_s - nb), n_s):
            out_desc(s, s % nb, osem[s % nb]).wait()

    return k(table, idx2)


def _subcore_ranges(total):
    rpt = -(-total // 16 // 8) * 8
    last = total - 15 * rpt
    assert last > 0 and last % 8 == 0
    return rpt, last


@functools.partial(jax.jit, static_argnames=("n_nodes", "d", "ch"))
def _sc_scatter_add(rows, idx6, zeros, *, n_nodes, d, ch=128):
    n_rows = rows.shape[0]
    half = n_nodes // 2
    acc_rows = half + 8
    per_t = n_rows // 16
    n_s = per_t // ch
    tail = per_t - n_s * ch
    n_chunks = n_s + (1 if tail else 0)
    rpt_i, last_i = _subcore_ranges(acc_rows)
    rpt_o, last_o = _subcore_ranges(half)
    mesh = plsc.VectorSubcoreMesh(core_axis_name="c", subcore_axis_name="s")

    scratch = ([pltpu.VMEM((n_chunks, ch), jnp.int32)]
               + [pltpu.VMEM((ch, d), jnp.float32) for _ in range(_NB)]
               + [pltpu.SemaphoreType.DMA for _ in range(_NB)]
               + [pltpu.VMEM_SHARED((acc_rows, d), jnp.float32)])

    @functools.partial(
        pl.kernel, mesh=mesh,
        out_type=jax.ShapeDtypeStruct((n_nodes, d), jnp.float32),
        scratch_types=scratch,
    )
    def k(rows_hbm, idx_hbm, zeros_hbm, out_hbm, idx_v, *rest):
        bufs = rest[:_NB]
        isem = rest[_NB:2 * _NB]
        acc = rest[2 * _NB]
        cid = lax.axis_index("c")
        sid = lax.axis_index("s")
        base = sid * per_t
        pltpu.sync_copy(idx_hbm.at[cid, sid], idx_v)

        @pl.when(sid < 15)
        def _():
            pltpu.sync_copy(zeros_hbm.at[pl.ds(sid * rpt_i, rpt_i)],
                            acc.at[pl.ds(sid * rpt_i, rpt_i)])

        @pl.when(sid == 15)
        def _():
            pltpu.sync_copy(zeros_hbm.at[pl.ds(15 * rpt_i, last_i)],
                            acc.at[pl.ds(15 * rpt_i, last_i)])

        plsc.subcore_barrier()

        if tail:
            tdesc = pltpu.make_async_copy(
                rows_hbm.at[pl.ds(base + n_s * ch, tail)],
                bufs[0].at[pl.ds(0, tail)], isem[0])
            tdesc.start()
            tdesc.wait()
            pltpu.sync_copy(bufs[0], acc.at[idx_v.at[n_s]], add=True)

        def in_desc(s, b):
            return pltpu.make_async_copy(
                rows_hbm.at[pl.ds(base + s * ch, ch)], bufs[b], isem[b])

        for b in range(_NB):
            if b < n_s:
                in_desc(b, b).start()

        n_outer = (n_s + _NB - 1) // _NB

        def outer(t, carry):
            for b in range(_NB):
                s = t * _NB + b

                @pl.when(s < n_s)
                def _():
                    in_desc(s, b).wait()
                    pltpu.sync_copy(bufs[b], acc.at[idx_v.at[s]], add=True)
                    sn = s + _NB

                    @pl.when(sn < n_s)
                    def _():
                        in_desc(sn, b).start()
            return carry

        lax.fori_loop(0, n_outer, outer, 0)
        plsc.subcore_barrier()

        @pl.when(sid < 15)
        def _():
            pltpu.sync_copy(acc.at[pl.ds(sid * rpt_o, rpt_o)],
                            out_hbm.at[pl.ds(cid * half + sid * rpt_o, rpt_o)])

        @pl.when(sid == 15)
        def _():
            pltpu.sync_copy(acc.at[pl.ds(15 * rpt_o, last_o)],
                            out_hbm.at[pl.ds(cid * half + 15 * rpt_o, last_o)])

    return k(rows, idx6, zeros)


@functools.partial(jax.jit, static_argnames=("n_rows", "n_nodes", "d", "ch"))
def _sc_count(idx6, ones_rows, zeros, *, n_rows, n_nodes, d, ch=128):
    half = n_nodes // 2
    acc_rows = half + 8
    per_t = n_rows // 16
    n_s = -(-per_t // ch)
    rpt_i, last_i = _subcore_ranges(acc_rows)
    rpt_o, last_o = _subcore_ranges(half)
    mesh = plsc.VectorSubcoreMesh(core_axis_name="c", subcore_axis_name="s")

    scratch = [pltpu.VMEM((n_s, ch), jnp.int32),
               pltpu.VMEM((ch, d), jnp.float32),
               pltpu.VMEM_SHARED((acc_rows, d), jnp.float32)]

    @functools.partial(
        pl.kernel, mesh=mesh,
        out_type=jax.ShapeDtypeStruct((n_nodes, d), jnp.float32),
        scratch_types=scratch,
    )
    def k(idx_hbm, ones_hbm, zeros_hbm, out_hbm, idx_v, ones_v, acc):
        cid = lax.axis_index("c")
        sid = lax.axis_index("s")
        pltpu.sync_copy(idx_hbm.at[cid, sid], idx_v)
        pltpu.sync_copy(ones_hbm, ones_v)

        @pl.when(sid < 15)
        def _():
            pltpu.sync_copy(zeros_hbm.at[pl.ds(sid * rpt_i, rpt_i)],
                            acc.at[pl.ds(sid * rpt_i, rpt_i)])

        @pl.when(sid == 15)
        def _():
            pltpu.sync_copy(zeros_hbm.at[pl.ds(15 * rpt_i, last_i)],
                            acc.at[pl.ds(15 * rpt_i, last_i)])

        plsc.subcore_barrier()

        def body(s, carry):
            pltpu.sync_copy(ones_v, acc.at[idx_v.at[s]], add=True)
            return carry

        lax.fori_loop(0, n_s, body, 0)
        plsc.subcore_barrier()

        @pl.when(sid < 15)
        def _():
            pltpu.sync_copy(acc.at[pl.ds(sid * rpt_o, rpt_o)],
                            out_hbm.at[pl.ds(cid * half + sid * rpt_o, rpt_o)])

        @pl.when(sid == 15)
        def _():
            pltpu.sync_copy(acc.at[pl.ds(15 * rpt_o, last_o)],
                            out_hbm.at[pl.ds(cid * half + 15 * rpt_o, last_o)])

    return k(idx6, ones_rows, zeros)


def _tc_dual_proj(h, wa, wb):
    n, din = h.shape
    da = wa.shape[1]
    db = wb.shape[1]
    blk = 2000

    def body(h_ref, wa_ref, wb_ref, oa_ref, ob_ref):
        hb = h_ref[...]
        oa_ref[...] = jnp.dot(hb, wa_ref[...],
                              preferred_element_type=jnp.float32)
        ob_ref[...] = jnp.dot(hb, wb_ref[...],
                              preferred_element_type=jnp.float32)

    return pl.pallas_call(
        body,
        grid=(n // blk,),
        in_specs=[pl.BlockSpec((blk, din), lambda i: (i, 0)),
                  pl.BlockSpec((din, da), lambda i: (0, 0)),
                  pl.BlockSpec((din, db), lambda i: (0, 0))],
        out_specs=[pl.BlockSpec((blk, da), lambda i: (i, 0)),
                   pl.BlockSpec((blk, db), lambda i: (i, 0))],
        out_shape=[jax.ShapeDtypeStruct((n, da), jnp.float32),
                   jax.ShapeDtypeStruct((n, db), jnp.float32)],
    )(h, wa, wb)


def _tc_msg_mlp(gd, gs, e, w1e, b1, w2, b2, w3, b3):
    n, d = gd.shape
    de = e.shape[1]
    blk = 2000

    def body(gd_ref, gs_ref, e_ref, w1e_ref, b1_ref, w2_ref, b2_ref,
             w3_ref, b3_ref, o_ref):
        t = gd_ref[...] + gs_ref[...] + jnp.dot(
            e_ref[...], w1e_ref[...], preferred_element_type=jnp.float32)
        t = jnp.maximum(t + b1_ref[...], 0.0)
        t = jnp.maximum(
            jnp.dot(t, w2_ref[...], preferred_element_type=jnp.float32)
            + b2_ref[...], 0.0)
        o_ref[...] = jnp.dot(
            t, w3_ref[...], preferred_element_type=jnp.float32) + b3_ref[...]

    full = pl.BlockSpec((1, d), lambda i: (0, 0))
    return pl.pallas_call(
        body,
        grid=(n // blk,),
        in_specs=[pl.BlockSpec((blk, d), lambda i: (i, 0)),
                  pl.BlockSpec((blk, d), lambda i: (i, 0)),
                  pl.BlockSpec((blk, de), lambda i: (i, 0)),
                  pl.BlockSpec((de, d), lambda i: (0, 0)),
                  full,
                  pl.BlockSpec((d, d), lambda i: (0, 0)),
                  full,
                  pl.BlockSpec((d, d), lambda i: (0, 0)),
                  full],
        out_specs=pl.BlockSpec((blk, d), lambda i: (i, 0)),
        out_shape=jax.ShapeDtypeStruct((n, d), jnp.float32),
    )(gd, gs, e, w1e, b1.reshape(1, d), w2, b2.reshape(1, d),
      w3, b3.reshape(1, d))


def _layer_norm(v, g, b):
    mu = jnp.mean(v, axis=-1, keepdims=True)
    var = jnp.mean((v - mu) ** 2, axis=-1, keepdims=True)
    return (v - mu) * lax.rsqrt(var + 1e-5) * g + b


def _tc_node_update(h, agg_a, agg_b, counts, ln1g, ln1b, wf1, bf1, wf2,
                    bf2, ln2g, ln2b):
    n, d = h.shape
    dh = wf1.shape[1]
    blk = 2000

    def body(h_ref, pa_ref, pb_ref, c_ref, ln1g_ref, ln1b_ref, wf1_ref,
             bf1_ref, wf2_ref, bf2_ref, ln2g_ref, ln2b_ref, o_ref):
        cnt = c_ref[...][:, 0:1]
        agg = (pa_ref[...] + pb_ref[...]) / jnp.maximum(cnt, 1.0)
        u = _layer_norm(h_ref[...] + agg, ln1g_ref[...], ln1b_ref[...])
        ff = jnp.maximum(
            jnp.dot(u, wf1_ref[...], preferred_element_type=jnp.float32)
            + bf1_ref[...], 0.0)
        ff = jnp.dot(ff, wf2_ref[...], preferred_element_type=jnp.float32) \
            + bf2_ref[...]
        o_ref[...] = _layer_norm(u + ff, ln2g_ref[...], ln2b_ref[...])

    vec = pl.BlockSpec((1, d), lambda i: (0, 0))
    return pl.pallas_call(
        body,
        grid=(n // blk,),
        in_specs=[pl.BlockSpec((blk, d), lambda i: (i, 0)),
                  pl.BlockSpec((blk, d), lambda i: (i, 0)),
                  pl.BlockSpec((blk, d), lambda i: (i, 0)),
                  pl.BlockSpec((blk, d), lambda i: (i, 0)),
                  vec, vec,
                  pl.BlockSpec((d, dh), lambda i: (0, 0)),
                  pl.BlockSpec((1, dh), lambda i: (0, 0)),
                  pl.BlockSpec((dh, d), lambda i: (0, 0)),
                  vec, vec, vec],
        out_specs=pl.BlockSpec((blk, d), lambda i: (i, 0)),
        out_shape=jax.ShapeDtypeStruct((n, d), jnp.float32),
    )(h, agg_a, agg_b, counts, ln1g.reshape(1, d), ln1b.reshape(1, d), wf1,
      bf1.reshape(1, dh), wf2, bf2.reshape(1, d), ln2g.reshape(1, d),
      ln2b.reshape(1, d))


def _tc_boundary_msg(hd, hs, e, eparams, lne, mparams):
    n, d = hd.shape
    de = e.shape[1]
    blk = 2000
    (u1, c1), (u2, c2), (u3, c3) = eparams
    lng, lnb = lne
    (w1, b1), (w2, b2), (w3, b3) = mparams
    u1s, u1d, u1e = u1[:d], u1[d:2 * d], u1[2 * d:]
    w1d, w1e, w1s = w1[:d], w1[d:d + de], w1[d + de:]

    def body(hd_ref, hs_ref, e_ref, u1s_ref, u1d_ref, u1e_ref, c1_ref,
             u2_ref, c2_ref, u3_ref, c3_ref, lng_ref, lnb_ref,
             w1d_ref, w1e_ref, w1s_ref, b1_ref, w2_ref, b2_ref,
             w3_ref, b3_ref, o_ref):
        hdv = hd_ref[...]
        hsv = hs_ref[...]
        ev = e_ref[...]

        def mm(a, w_ref):
            return jnp.dot(a, w_ref[...], preferred_element_type=jnp.float32)

        t = mm(hsv, u1s_ref) + mm(hdv, u1d_ref) + mm(ev, u1e_ref)
        t = jnp.maximum(t + c1_ref[...], 0.0)
        t = jnp.maximum(mm(t, u2_ref) + c2_ref[...], 0.0)
        t = mm(t, u3_ref) + c3_ref[...]
        e2 = _layer_norm(ev + t, lng_ref[...], lnb_ref[...])

        m = mm(hdv, w1d_ref) + mm(hsv, w1s_ref) + mm(e2, w1e_ref)
        m = jnp.maximum(m + b1_ref[...], 0.0)
        m = jnp.maximum(mm(m, w2_ref) + b2_ref[...], 0.0)
        o_ref[...] = mm(m, w3_ref) + b3_ref[...]

    hb = pl.BlockSpec((blk, d), lambda i: (i, 0))
    eb = pl.BlockSpec((blk, de), lambda i: (i, 0))
    p128 = pl.BlockSpec((d, 128), lambda i: (0, 0))
    p16 = pl.BlockSpec((d, de), lambda i: (0, 0))
    sq = pl.BlockSpec((de, de), lambda i: (0, 0))
    v16 = pl.BlockSpec((1, de), lambda i: (0, 0))
    v128 = pl.BlockSpec((1, d), lambda i: (0, 0))
    return pl.pallas_call(
        body,
        grid=(n // blk,),
        in_specs=[hb, hb, eb,
                  p16, p16, sq, v16,
                  sq, v16, sq, v16,
                  v16, v16,
                  p128, pl.BlockSpec((de, d), lambda i: (0, 0)), p128, v128,
                  pl.BlockSpec((d, d), lambda i: (0, 0)), v128,
                  pl.BlockSpec((d, d), lambda i: (0, 0)), v128],
        out_specs=hb,
        out_shape=jax.ShapeDtypeStruct((n, d), jnp.float32),
    )(hd, hs, e, u1s, u1d, u1e, c1.reshape(1, de), u2, c2.reshape(1, de),
      u3, c3.reshape(1, de), lng.reshape(1, de), lnb.reshape(1, de),
      w1d, w1e, w1s, b1.reshape(1, d), w2, b2.reshape(1, d),
      w3, b3.reshape(1, d))


def kernel(x, edge_index, edge_attr, params):
    n_nodes = x.shape[1]
    d = x.shape[2]
    de = edge_attr.shape[1]
    n_edges = edge_index.shape[1]
    h = x.reshape(n_nodes, d)
    e = edge_attr
    src = edge_index[0].astype(jnp.int32)
    dst = edge_index[1].astype(jnp.int32)
    eh = n_edges // 2
    per_w = eh // _NW
    src2 = [src[hb * eh:(hb + 1) * eh].reshape(_NW, per_w) for hb in range(2)]
    dst2 = [dst[hb * eh:(hb + 1) * eh].reshape(_NW, per_w) for hb in range(2)]
    ehalf = [e[:eh], e[eh:]]

    half = n_nodes // 2
    dst_lo = jnp.where(dst < half, dst, half)
    dst_hi = jnp.where(dst >= half, dst - half, half)
    sch = 128
    per_t = eh // 16
    n_chunks = -(-per_t // sch)
    pad = n_chunks * sch - per_t

    def mk_idx6(hb):
        a = jnp.stack([dst_lo[hb * eh:(hb + 1) * eh],
                       dst_hi[hb * eh:(hb + 1) * eh]]).reshape(2, 16, per_t)
        a = jnp.pad(a, ((0, 0), (0, 0), (0, pad)), constant_values=half)
        return a.reshape(2, 16, n_chunks, sch)

    idx6 = [mk_idx6(0), mk_idx6(1)]
    zeros_acc = jnp.zeros((half + 8, d), jnp.float32)
    ones_rows = jnp.ones((sch, d), jnp.float32)

    layers = params["layers"]
    n_layers = len(layers)
    mrows = counts = None
    for li, p in enumerate(layers):
        if mrows is None:
            (w1, b1), (w2, b2), (w3, b3) = p["msg"]
            w1d, w1e, w1s = w1[:d], w1[d:d + de], w1[d + de:]
            hd, hs = _tc_dual_proj(h, w1d, w1s)
            g = [(_sc_gather(hd, dst2[hb], n_rows=eh, d=d),
                  _sc_gather(hs, src2[hb], n_rows=eh, d=d))
                 for hb in range(2)]
            idx6c_b, _ = lax.optimization_barrier((idx6[0], g[0][1]))
            counts = [_sc_count(idx6c_b, ones_rows, zeros_acc, n_rows=eh,
                                n_nodes=n_nodes, d=d, ch=sch)]
            idx6c2_b, _ = lax.optimization_barrier((idx6[1], g[1][1]))
            counts.append(_sc_count(idx6c2_b, ones_rows, zeros_acc,
                                    n_rows=eh, n_nodes=n_nodes, d=d, ch=sch))
            mrows = [_tc_msg_mlp(g[hb][0], g[hb][1], ehalf[hb],
                                 w1e, b1, w2, b2, w3, b3) for hb in range(2)]
        aggs = [_sc_scatter_add(mrows[hb], idx6[hb], zeros_acc,
                                n_nodes=n_nodes, d=d) for hb in range(2)]
        if isinstance(counts, list):
            counts = counts[0] + counts[1]
        (wf1, bf1), (wf2, bf2) = p["ff"]
        h = _tc_node_update(h, aggs[0], aggs[1], counts,
                            p["ln1"][0], p["ln1"][1],
                            wf1, bf1, wf2, bf2, p["ln2"][0], p["ln2"][1])
        if li < n_layers - 1:
            r = [(_sc_gather(h, dst2[hb], n_rows=eh, d=d),
                  _sc_gather(h, src2[hb], n_rows=eh, d=d))
                 for hb in range(2)]
            mrows = [_tc_boundary_msg(r[hb][0], r[hb][1], ehalf[hb],
                                      p["edge"], p["ln_e"],
                                      layers[li + 1]["msg"])
                     for hb in range(2)]
    return h.reshape(x.shape)
